# Initial kernel scaffold; baseline (speedup 1.0000x reference)
import jax
import jax.numpy as jnp
from jax.experimental import pallas as pl


def kernel(x, edge_index, mmse, Wl1, Wr1, a1, b1, Wl2, Wr2, a2, b2, Wm, bm, W11, b11, W12, b12, W21, b21, W22, b22, Wp, bp):
    raise NotImplementedError("write your pallas kernel here")


# speedup vs baseline: 8.1526x; 8.1526x over previous
import jax
import jax.numpy as jnp
from jax.experimental import pallas as pl

_N, _E, _T, _D = 19, 342, 1025, 512
_NP, _EP, _TP = 32, 384, 1152
_NEG = -1e30


def _leaky(z):
    return jnp.where(z > 0, z, 0.2 * z)


def _attn_layer(x, srcf, dstf, dmask, wl, wr, a_row, b_row):
    xl = jnp.dot(x, wl, preferred_element_type=jnp.float32)
    xr = jnp.dot(x, wr, preferred_element_type=jnp.float32)
    he = _leaky(jnp.dot(srcf, xl, preferred_element_type=jnp.float32)
                + jnp.dot(dstf, xr, preferred_element_type=jnp.float32))
    e_row = jax.lax.dot_general(
        a_row, he, (((1,), (1,)), ((), ())),
        preferred_element_type=jnp.float32)
    eb = jnp.broadcast_to(e_row, (_NP, _EP))
    m = jnp.max(jnp.where(dmask, eb, _NEG), axis=1, keepdims=True)
    mdst = jnp.max(jnp.where(dmask, jnp.broadcast_to(m, (_NP, _EP)), _NEG),
                   axis=0, keepdims=True)
    ex = jnp.exp(e_row - mdst)
    s = jnp.sum(jnp.where(dmask, jnp.broadcast_to(ex, (_NP, _EP)), 0.0),
                axis=1, keepdims=True)
    sdst = jnp.sum(jnp.where(dmask, jnp.broadcast_to(s, (_NP, _EP)), 0.0),
                   axis=0, keepdims=True)
    alpha = ex / (sdst + 1e-16)
    aw = jnp.where(dmask, jnp.broadcast_to(alpha, (_NP, _EP)), 0.0)
    adj = jnp.dot(aw, srcf, preferred_element_type=jnp.float32)
    h = jnp.dot(adj, xl, preferred_element_type=jnp.float32) + b_row
    return h, alpha


def _fused(src_ref, dstc_ref, dstr_ref, x_ref, wl1_ref, wr1_ref, a1_ref, b1_ref,
           wl2_ref, wr2_ref, a2_ref, b2_ref, mmse_ref, wm_ref, bm_ref,
           dec_ref, al_ref):
    ioncol = jax.lax.broadcasted_iota(jnp.int32, (_EP, _NP), 1)
    srcf = (src_ref[:] == ioncol).astype(jnp.float32)
    dstf = (dstc_ref[:] == ioncol).astype(jnp.float32)
    dmask = jax.lax.broadcasted_iota(jnp.int32, (_NP, _EP), 0) == dstr_ref[:]
    h1, alpha1 = _attn_layer(x_ref[:], srcf, dstf, dmask,
                             wl1_ref[:], wr1_ref[:], a1_ref[:], b1_ref[:])
    h2, _ = _attn_layer(h1, srcf, dstf, dmask,
                        wl2_ref[:], wr2_ref[:], a2_ref[:], b2_ref[:])
    gf = h2 + (mmse_ref[0, 0] * wm_ref[:] + bm_ref[:])
    dec = jax.lax.dot_general(gf, gf, (((1,), (1,)), ((), ())),
                              preferred_element_type=jnp.float32)
    dec_ref[:] = jax.nn.sigmoid(dec)
    al_ref[:] = alpha1


def kernel(x, edge_index, mmse, Wl1, Wr1, a1, b1, Wl2, Wr2, a2, b2, Wm, bm,
           W11, b11, W12, b12, W21, b21, W22, b22, Wp, bp):
    src = edge_index[0].astype(jnp.int32)
    dst = edge_index[1].astype(jnp.int32)
    src_col = jnp.zeros((_EP, 1), jnp.int32).at[:_E, 0].set(src)
    dst_col = jnp.full((_EP, 1), _NP - 1, jnp.int32).at[:_E, 0].set(dst)
    dst_row = dst_col.T
    xp = jnp.zeros((_NP, _TP), jnp.float32).at[:_N, :_T].set(x)
    wl1 = jnp.zeros((_TP, _D), jnp.float32).at[:_T].set(Wl1)
    wr1 = jnp.zeros((_TP, _D), jnp.float32).at[:_T].set(Wr1)
    dec, al = pl.pallas_call(
        _fused,
        out_shape=[jax.ShapeDtypeStruct((_NP, _NP), jnp.float32),
                   jax.ShapeDtypeStruct((1, _EP), jnp.float32)],
    )(src_col, dst_col, dst_row, xp, wl1, wr1,
      a1.reshape(1, _D), b1.reshape(1, _D), Wl2, Wr2,
      a2.reshape(1, _D), b2.reshape(1, _D), mmse.reshape(1, 1),
      Wm, bm.reshape(1, _D))
    return dec[:_N, :_N], al[0, :_E]


# speedup vs baseline: 13.7277x; 1.6838x over previous
import jax
import jax.numpy as jnp
from jax.experimental import pallas as pl

_N, _E, _T, _D = 19, 342, 1025, 512
_EP = 384
_PAD_ID = 31
_NEG = -1e30


def _leaky(z):
    return jnp.where(z > 0, z, 0.2 * z)


def _attn_layer(x, srcf, dstf, dmask, wl, wr, a_row, b_row):
    xl = jnp.dot(x, wl, preferred_element_type=jnp.float32)
    xr = jnp.dot(x, wr, preferred_element_type=jnp.float32)
    he = _leaky(jnp.dot(srcf, xl, preferred_element_type=jnp.float32)
                + jnp.dot(dstf, xr, preferred_element_type=jnp.float32))
    e_row = jax.lax.dot_general(
        a_row, he, (((1,), (1,)), ((), ())),
        preferred_element_type=jnp.float32)
    eb = jnp.broadcast_to(e_row, (_N, _EP))
    m = jnp.max(jnp.where(dmask, eb, _NEG), axis=1, keepdims=True)
    mdst = jnp.max(jnp.where(dmask, jnp.broadcast_to(m, (_N, _EP)), _NEG),
                   axis=0, keepdims=True)
    ex = jnp.exp(jnp.minimum(e_row - mdst, 0.0))
    s = jnp.sum(jnp.where(dmask, jnp.broadcast_to(ex, (_N, _EP)), 0.0),
                axis=1, keepdims=True)
    sdst = jnp.sum(jnp.where(dmask, jnp.broadcast_to(s, (_N, _EP)), 0.0),
                   axis=0, keepdims=True)
    alpha = ex / (sdst + 1e-16)
    aw = jnp.where(dmask, jnp.broadcast_to(alpha, (_N, _EP)), 0.0)
    adj = jnp.dot(aw, srcf, preferred_element_type=jnp.float32)
    h = jnp.dot(adj, xl, preferred_element_type=jnp.float32) + b_row
    return h, alpha


def _fused(src_ref, dstc_ref, dstr_ref, x_ref, wl1_ref, wr1_ref, a1_ref, b1_ref,
           wl2_ref, wr2_ref, a2_ref, b2_ref, mmse_ref, wm_ref, bm_ref,
           dec_ref, al_ref):
    ioncol = jax.lax.broadcasted_iota(jnp.int32, (_EP, _N), 1)
    srcf = (src_ref[:] == ioncol).astype(jnp.float32)
    dstf = (dstc_ref[:] == ioncol).astype(jnp.float32)
    dmask = jax.lax.broadcasted_iota(jnp.int32, (_N, _EP), 0) == dstr_ref[:]
    h1, alpha1 = _attn_layer(x_ref[:], srcf, dstf, dmask,
                             wl1_ref[:], wr1_ref[:], a1_ref[:], b1_ref[:])
    h2, _ = _attn_layer(h1, srcf, dstf, dmask,
                        wl2_ref[:], wr2_ref[:], a2_ref[:], b2_ref[:])
    gf = h2 + (mmse_ref[0, 0] * wm_ref[:] + bm_ref[:])
    dec = jax.lax.dot_general(gf, gf, (((1,), (1,)), ((), ())),
                              preferred_element_type=jnp.float32)
    dec_ref[:] = jax.nn.sigmoid(dec)
    al_ref[:] = alpha1[:, :_E]


def kernel(x, edge_index, mmse, Wl1, Wr1, a1, b1, Wl2, Wr2, a2, b2, Wm, bm,
           W11, b11, W12, b12, W21, b21, W22, b22, Wp, bp):
    src = edge_index[0].astype(jnp.int32)
    dst = edge_index[1].astype(jnp.int32)
    src_col = jnp.full((_EP, 1), _PAD_ID, jnp.int32).at[:_E, 0].set(src)
    dst_col = jnp.full((_EP, 1), _PAD_ID, jnp.int32).at[:_E, 0].set(dst)
    dst_row = dst_col.T
    dec, al = pl.pallas_call(
        _fused,
        out_shape=[jax.ShapeDtypeStruct((_N, _N), jnp.float32),
                   jax.ShapeDtypeStruct((1, _E), jnp.float32)],
    )(src_col, dst_col, dst_row, x, Wl1, Wr1,
      a1.reshape(1, _D), b1.reshape(1, _D), Wl2, Wr2,
      a2.reshape(1, _D), b2.reshape(1, _D), mmse.reshape(1, 1),
      Wm, bm.reshape(1, _D))
    return dec, al[0]


# speedup vs baseline: 18.6575x; 1.3591x over previous
import jax
import jax.numpy as jnp
from jax.experimental import pallas as pl

_N, _E, _T, _D = 19, 342, 1025, 512
_NEG = -1e30


def _leaky(z):
    return jnp.where(z > 0, z, 0.2 * z)


def _attn_layer(x, srcf, dstf, dmask, wl, wr, a_row, b_row):
    xl = jnp.dot(x, wl, preferred_element_type=jnp.float32)
    xr = jnp.dot(x, wr, preferred_element_type=jnp.float32)
    xls = jax.lax.dot_general(srcf, xl, (((0,), (0,)), ((), ())),
                              preferred_element_type=jnp.float32)
    xrd = jax.lax.dot_general(dstf, xr, (((0,), (0,)), ((), ())),
                              preferred_element_type=jnp.float32)
    he = _leaky(xls + xrd)
    e_row = jax.lax.dot_general(a_row, he, (((1,), (1,)), ((), ())),
                                preferred_element_type=jnp.float32)
    eb = jnp.broadcast_to(e_row, (_N, _E))
    m = jnp.max(jnp.where(dmask, eb, _NEG), axis=1, keepdims=True)
    mdst = jnp.max(jnp.where(dmask, jnp.broadcast_to(m, (_N, _E)), _NEG),
                   axis=0, keepdims=True)
    ex = jnp.exp(jnp.minimum(e_row - mdst, 0.0))
    s = jnp.sum(jnp.where(dmask, jnp.broadcast_to(ex, (_N, _E)), 0.0),
                axis=1, keepdims=True)
    sdst = jnp.sum(jnp.where(dmask, jnp.broadcast_to(s, (_N, _E)), 0.0),
                   axis=0, keepdims=True)
    alpha = ex / (sdst + 1e-16)
    aw = jnp.where(dmask, jnp.broadcast_to(alpha, (_N, _E)), 0.0)
    adj = jax.lax.dot_general(aw, srcf, (((1,), (1,)), ((), ())),
                              preferred_element_type=jnp.float32)
    h = jnp.dot(adj, xl, preferred_element_type=jnp.float32) + b_row
    return h, alpha


def _fused(ei_ref, x_ref, wl1_ref, wr1_ref, a1_ref, b1_ref,
           wl2_ref, wr2_ref, a2_ref, b2_ref, mmse_ref, wm_ref, bm_ref,
           dec_ref, al_ref):
    src_row = ei_ref[0:1, :]
    dst_row = ei_ref[1:2, :]
    ion = jax.lax.broadcasted_iota(jnp.int32, (_N, _E), 0)
    srcf = (ion == src_row).astype(jnp.float32)
    dmask = ion == dst_row
    dstf = dmask.astype(jnp.float32)
    a1_row = a1_ref[:].reshape(1, _D)
    a2_row = a2_ref[:].reshape(1, _D)
    h1, alpha1 = _attn_layer(x_ref[:], srcf, dstf, dmask,
                             wl1_ref[:], wr1_ref[:], a1_row,
                             b1_ref[:].reshape(1, _D))
    h2, _ = _attn_layer(h1, srcf, dstf, dmask,
                        wl2_ref[:], wr2_ref[:], a2_row,
                        b2_ref[:].reshape(1, _D))
    gf = h2 + (mmse_ref[0] * wm_ref[:] + bm_ref[:].reshape(1, _D))
    dec = jax.lax.dot_general(gf, gf, (((1,), (1,)), ((), ())),
                              preferred_element_type=jnp.float32)
    dec_ref[:] = jax.nn.sigmoid(dec)
    al_ref[:] = alpha1


def kernel(x, edge_index, mmse, Wl1, Wr1, a1, b1, Wl2, Wr2, a2, b2, Wm, bm,
           W11, b11, W12, b12, W21, b21, W22, b22, Wp, bp):
    dec, al = pl.pallas_call(
        _fused,
        out_shape=[jax.ShapeDtypeStruct((_N, _N), jnp.float32),
                   jax.ShapeDtypeStruct((1, _E), jnp.float32)],
    )(edge_index, x, Wl1, Wr1, a1, b1, Wl2, Wr2, a2, b2, mmse, Wm, bm)
    return dec, al[0]
